# baseline (speedup 1.0000x reference)
import jax
import jax.numpy as jnp
from jax import lax
from jax.experimental import pallas as pl
from jax.experimental.pallas import tpu as pltpu
from jax.experimental.pallas import tpu_sc as plsc

_NC = 2
_NS = 16
_NW = _NC * _NS
_L = 16

_BATCH = 16384
_COMP = 4
_DIMS = 64
_PAIR = 128
_T0 = 1000
_T1 = 1000
_T1P = 1024
_NBLK = _T1P // _PAIR
_NPAIR = _T0 * _T1P // 2
_UPW = _T0 * _NBLK // _NW

_BPW = _BATCH // _NW
_RPW = _BPW * _COMP
_CHUNK_R = 128
_CHUNK_B = _CHUNK_R // _COMP
_NCH = _RPW // _CHUNK_R
_RING = 4


def _relayout_body(nat_hbm, tail_hbm, out_hbm, src0, src1, dst0, dst1,
                   ssem0, ssem1, dsem0, dsem1):
    wid = lax.axis_index("s") * _NC + lax.axis_index("c")
    tb = lax.shift_right_logical(wid, 2)
    t0w = jnp.bitwise_and(wid, 3)
    srcs = (src0, src1)
    dsts = (dst0, dst1)
    ssems = (ssem0, ssem1)
    dsems = (dsem0, dsem1)

    iotav = lax.iota(jnp.int32, _L)
    dvecs = [iotav + _L * k4 for k4 in range(_DIMS // _L)]

    def unit_t0(k):
        return t0w + 4 * k

    def unit_out(k):
        return (unit_t0(k) * _NBLK + tb) * (_PAIR // 2)

    def run():
        npair = _PAIR // 2

        def src_slice(k):
            return nat_hbm.at[unit_t0(k), :, pl.ds(tb * _PAIR, _PAIR)]

        def issue_src(k, b):
            return pltpu.async_copy(src_slice(k), srcs[b].at[pl.ds(0, _DIMS)],
                                    ssems[b])

        def wait_src(k, b):
            pltpu.make_async_copy(src_slice(k), srcs[b].at[pl.ds(0, _DIMS)],
                                  ssems[b]).wait()

        def issue_dst(k, b):
            return pltpu.async_copy(
                dsts[b], out_hbm.at[pl.ds(unit_out(k), _PAIR // 2)],
                dsems[b])

        def wait_dst(k, b):
            pltpu.make_async_copy(
                dsts[b], out_hbm.at[pl.ds(unit_out(k), _PAIR // 2)],
                dsems[b]).wait()

        def transpose(b):
            sbuf = srcs[b]
            dbuf = dsts[b]

            def prow(p, carry):
                for h in range(2):
                    cvec = jnp.full((_L,), 2 * p + h, jnp.int32)
                    for k4 in range(_DIMS // _L):
                        v = plsc.load_gather(sbuf, [dvecs[k4], cvec])
                        dbuf[p, pl.ds(h * _DIMS + k4 * _L, _L)] = v
                return carry

            lax.fori_loop(0, npair, prow, 0)

        issue_src(0, 0)
        issue_src(1, 1)
        wait_src(0, 0)
        transpose(0)
        issue_src(2, 0)
        issue_dst(0, 0)
        wait_src(1, 1)
        transpose(1)
        issue_src(3, 1)
        issue_dst(1, 1)

        def outer(kk, carry):
            for b in range(2):
                k = 2 * kk + b
                wait_dst(k - 2, b)
                wait_src(k, b)
                transpose(b)
                issue_src(k + 2, b)
                issue_dst(k, b)
            return carry

        lax.fori_loop(1, _UPW // 2 - 1, outer, 0)

        for b in range(2):
            k = _UPW - 2 + b
            wait_dst(k - 2, b)
            wait_src(k, b)
            transpose(b)
            issue_dst(k, b)
        for b in range(2):
            wait_dst(_UPW - 2 + b, b)

    def run_tail():
        def issue_in(k, b):
            return pltpu.async_copy(tail_hbm.at[unit_t0(k)],
                                    srcs[b].at[pl.ds(0, _PAIR // 2)],
                                    ssems[b])

        def wait_in(k, b):
            pltpu.make_async_copy(tail_hbm.at[unit_t0(k)],
                                  srcs[b].at[pl.ds(0, _PAIR // 2)],
                                  ssems[b]).wait()

        def issue_out(k, b):
            return pltpu.async_copy(
                srcs[b].at[pl.ds(12, 56)],
                out_hbm.at[pl.ds(unit_out(k), 56)], dsems[b])

        def wait_out(k, b):
            pltpu.make_async_copy(
                srcs[b].at[pl.ds(12, 56)],
                out_hbm.at[pl.ds(unit_out(k), 56)], dsems[b]).wait()

        issue_in(0, 0)
        issue_in(1, 1)

        def outer(kk, carry):
            for b in range(2):
                k = 2 * kk + b

                @pl.when(k >= 2)
                def _():
                    wait_out(k - 2, b)

                wait_in(k, b)

                @pl.when(k + 2 < _UPW)
                def _():
                    issue_in(k + 2, b)

                issue_out(k, b)
            return carry

        lax.fori_loop(0, _UPW // 2, outer, 0)
        for b in range(2):
            wait_out(_UPW - 2 + b, b)

    @pl.when(tb != _NBLK - 1)
    def _():
        run()

    @pl.when(tb == _NBLK - 1)
    def _():
        run_tail()


def _lookup_body(table_hbm, idx_hbm, w_hbm, out_hbm,
                 idx_v, ihi2, poff_v, w_v,
                 rows0, rows1, rows2, rows3,
                 outb0, outb1, outb2, outb3,
                 gsem0, gsem1, gsem2, gsem3,
                 osem0, osem1, osem2, osem3):
    wid = lax.axis_index("s") * _NC + lax.axis_index("c")
    row_base = wid * _RPW
    b_base = wid * _BPW

    rows = (rows0, rows1, rows2, rows3)
    outs = (outb0, outb1, outb2, outb3)
    gsems = (gsem0, gsem1, gsem2, gsem3)
    osems = (osem0, osem1, osem2, osem3)

    pltpu.sync_copy(idx_hbm.at[pl.ds(row_base, _RPW)], idx_v)

    def split(g, carry):
        sl = pl.ds(g * _L, _L)
        iv = idx_v[sl]
        poff_v[sl] = lax.shift_left(jnp.bitwise_and(iv, 1), 6)
        return carry

    lax.fori_loop(0, _RPW // _L, split, 0)

    def split2(c2, carry):
        def inner(l, carry2):
            ihi2[c2, pl.ds(l * _L, _L)] = lax.shift_right_logical(
                idx_v[pl.ds(c2 * _CHUNK_R + l * _L, _L)], 1)
            return carry2
        return lax.fori_loop(0, _CHUNK_R // _L, inner, carry)

    lax.fori_loop(0, _NCH, split2, 0)

    gdesc = [None] * _NCH
    for p in range(_RING - 1):
        gdesc[p] = pltpu.async_copy(
            table_hbm.at[ihi2.at[p]], rows[p], gsems[p])

    pltpu.sync_copy(w_hbm.at[pl.ds(row_base, _RPW)], w_v)

    odesc = [None] * _NCH
    for c in range(_NCH):
        nxt = c + _RING - 1
        if nxt < _NCH:
            gdesc[nxt] = pltpu.async_copy(
                table_hbm.at[ihi2.at[nxt]], rows[nxt % _RING],
                gsems[nxt % _RING])
        gdesc[c].wait()
        if c >= _RING:
            odesc[c - _RING].wait()

        rbuf = rows[c % _RING]
        obuf = outs[c % _RING]
        woff = c * _CHUNK_R

        def body(g, carry, rbuf=rbuf, obuf=obuf, woff=woff):
            wsl = pl.ds(woff + g * _L, _L)
            wv = w_v[wsl]
            pv = poff_v[wsl]
            for j in range(_L // _COMP):
                e = g * (_L // _COMP) + j
                rb = e * _COMP
                o0 = pv[_COMP * j]
                o1 = pv[_COMP * j + 1]
                o2 = pv[_COMP * j + 2]
                o3 = pv[_COMP * j + 3]
                w0 = wv[_COMP * j]
                w1 = wv[_COMP * j + 1]
                w2 = wv[_COMP * j + 2]
                w3 = wv[_COMP * j + 3]
                for s in range(_DIMS // _L):
                    acc = (rbuf[rb, pl.ds(o0 + s * _L, _L)] * w0
                           + rbuf[rb + 1, pl.ds(o1 + s * _L, _L)] * w1
                           + rbuf[rb + 2, pl.ds(o2 + s * _L, _L)] * w2
                           + rbuf[rb + 3, pl.ds(o3 + s * _L, _L)] * w3)
                    obuf[e, pl.ds(s * _L, _L)] = acc
            return carry

        lax.fori_loop(0, _CHUNK_R // _L, body, 0)

        odesc[c] = pltpu.async_copy(
            obuf, out_hbm.at[pl.ds(b_base + c * _CHUNK_B, _CHUNK_B)],
            osems[c % _RING])

    for c in range(_NCH - _RING, _NCH):
        odesc[c].wait()


@jax.jit
def _sc_lookup(nat, tail, idx, w):
    mesh = plsc.VectorSubcoreMesh(core_axis_name="c", subcore_axis_name="s")
    relayout = pl.kernel(
        _relayout_body,
        out_type=jax.ShapeDtypeStruct((_NPAIR, _PAIR), jnp.float32),
        mesh=mesh,
        scratch_types=[
            pltpu.VMEM((_DIMS + 8, _PAIR), jnp.float32),
            pltpu.VMEM((_DIMS + 8, _PAIR), jnp.float32),
            pltpu.VMEM((_PAIR // 2, _PAIR), jnp.float32),
            pltpu.VMEM((_PAIR // 2, _PAIR), jnp.float32),
            pltpu.SemaphoreType.DMA,
            pltpu.SemaphoreType.DMA,
            pltpu.SemaphoreType.DMA,
            pltpu.SemaphoreType.DMA,
        ],
        compiler_params=pltpu.CompilerParams(needs_layout_passes=False),
    )
    table = relayout(nat, tail)

    lookup = pl.kernel(
        _lookup_body,
        out_type=jax.ShapeDtypeStruct((_BATCH, _DIMS), jnp.float32),
        mesh=mesh,
        scratch_types=[
            pltpu.VMEM((_RPW,), jnp.int32),
            pltpu.VMEM((_NCH, _CHUNK_R), jnp.int32),
            pltpu.VMEM((_RPW,), jnp.int32),
            pltpu.VMEM((_RPW,), jnp.float32),
            pltpu.VMEM((_CHUNK_R, _PAIR), jnp.float32),
            pltpu.VMEM((_CHUNK_R, _PAIR), jnp.float32),
            pltpu.VMEM((_CHUNK_R, _PAIR), jnp.float32),
            pltpu.VMEM((_CHUNK_R, _PAIR), jnp.float32),
            pltpu.VMEM((_CHUNK_B, _DIMS), jnp.float32),
            pltpu.VMEM((_CHUNK_B, _DIMS), jnp.float32),
            pltpu.VMEM((_CHUNK_B, _DIMS), jnp.float32),
            pltpu.VMEM((_CHUNK_B, _DIMS), jnp.float32),
            pltpu.SemaphoreType.DMA,
            pltpu.SemaphoreType.DMA,
            pltpu.SemaphoreType.DMA,
            pltpu.SemaphoreType.DMA,
            pltpu.SemaphoreType.DMA,
            pltpu.SemaphoreType.DMA,
            pltpu.SemaphoreType.DMA,
            pltpu.SemaphoreType.DMA,
        ],
    )
    return lookup(table, idx, w)


def kernel(coords, coord_weights, embeddings):
    nat = embeddings.transpose(0, 2, 1)
    tail = embeddings[:, _T1 - _PAIR:, :].reshape(_T0, _PAIR // 2, 2 * _DIMS)
    idx = (coords[..., 0].astype(jnp.int32) * _T1P
           + coords[..., 1].astype(jnp.int32)).reshape(-1)
    w = coord_weights.reshape(-1)
    return _sc_lookup(nat, tail, idx, w)


# speedup vs baseline: 1.2116x; 1.2116x over previous
import jax
import jax.numpy as jnp
from jax import lax
from jax.experimental import pallas as pl
from jax.experimental.pallas import tpu as pltpu
from jax.experimental.pallas import tpu_sc as plsc

_NC = 2
_NS = 16
_NW = _NC * _NS
_L = 16

_BATCH = 16384
_COMP = 4
_DIMS = 64
_PAIR = 128
_T0 = 1000
_T1 = 1000
_T1P = 1024
_NBLK = _T1P // _PAIR
_NPAIR = _T0 * _T1P // 2
_UPW = _T0 * _NBLK // _NW

_BPW = _BATCH // _NW
_RPW = _BPW * _COMP
_CHUNK_R = 128
_CHUNK_B = _CHUNK_R // _COMP
_NCH = _RPW // _CHUNK_R
_RING = 4


def _relayout_body(nat_hbm, tail_hbm, out_hbm, src0, src1, dst0, dst1,
                   ssem0, ssem1, dsem0, dsem1):
    wid = lax.axis_index("s") * _NC + lax.axis_index("c")
    tb = lax.shift_right_logical(wid, 2)
    t0w = jnp.bitwise_and(wid, 3)
    srcs = (src0, src1)
    dsts = (dst0, dst1)
    ssems = (ssem0, ssem1)
    dsems = (dsem0, dsem1)

    iotav = lax.iota(jnp.int32, _L)
    rowvecs = [lax.shift_right_logical(iotav + _L * t, 1)
               for t in range(_PAIR // _L)]
    cvec0 = lax.shift_left(jnp.bitwise_and(iotav, 1), 6)

    def unit_t0(k):
        return t0w + 4 * k

    def unit_out(k):
        return (unit_t0(k) * _NBLK + tb) * (_PAIR // 2)

    def run():
        npair = _PAIR // 2

        def src_slice(k):
            return nat_hbm.at[unit_t0(k), :, pl.ds(tb * _PAIR, _PAIR)]

        def issue_src(k, b):
            return pltpu.async_copy(src_slice(k), srcs[b].at[pl.ds(0, _DIMS)],
                                    ssems[b])

        def wait_src(k, b):
            pltpu.make_async_copy(src_slice(k), srcs[b].at[pl.ds(0, _DIMS)],
                                  ssems[b]).wait()

        def issue_dst(k, b):
            return pltpu.async_copy(
                dsts[b], out_hbm.at[pl.ds(unit_out(k), _PAIR // 2)],
                dsems[b])

        def wait_dst(k, b):
            pltpu.make_async_copy(
                dsts[b], out_hbm.at[pl.ds(unit_out(k), _PAIR // 2)],
                dsems[b]).wait()

        def transpose(b):
            sbuf = srcs[b]
            dbuf = dsts[b]

            def drow(d, carry):
                colv = cvec0 + d
                for t in range(_PAIR // _L):
                    v = sbuf[d, pl.ds(t * _L, _L)]
                    plsc.store_scatter(dbuf, [rowvecs[t], colv], v)
                return carry

            lax.fori_loop(0, _DIMS, drow, 0)

        issue_src(0, 0)
        issue_src(1, 1)
        wait_src(0, 0)
        transpose(0)
        issue_src(2, 0)
        issue_dst(0, 0)
        wait_src(1, 1)
        transpose(1)
        issue_src(3, 1)
        issue_dst(1, 1)

        def outer(kk, carry):
            for b in range(2):
                k = 2 * kk + b
                wait_dst(k - 2, b)
                wait_src(k, b)
                transpose(b)
                issue_src(k + 2, b)
                issue_dst(k, b)
            return carry

        lax.fori_loop(1, _UPW // 2 - 1, outer, 0)

        for b in range(2):
            k = _UPW - 2 + b
            wait_dst(k - 2, b)
            wait_src(k, b)
            transpose(b)
            issue_dst(k, b)
        for b in range(2):
            wait_dst(_UPW - 2 + b, b)

    def run_tail():
        def issue_in(k, b):
            return pltpu.async_copy(tail_hbm.at[unit_t0(k)],
                                    srcs[b].at[pl.ds(0, _PAIR // 2)],
                                    ssems[b])

        def wait_in(k, b):
            pltpu.make_async_copy(tail_hbm.at[unit_t0(k)],
                                  srcs[b].at[pl.ds(0, _PAIR // 2)],
                                  ssems[b]).wait()

        def issue_out(k, b):
            return pltpu.async_copy(
                srcs[b].at[pl.ds(12, 56)],
                out_hbm.at[pl.ds(unit_out(k), 56)], dsems[b])

        def wait_out(k, b):
            pltpu.make_async_copy(
                srcs[b].at[pl.ds(12, 56)],
                out_hbm.at[pl.ds(unit_out(k), 56)], dsems[b]).wait()

        issue_in(0, 0)
        issue_in(1, 1)

        def outer(kk, carry):
            for b in range(2):
                k = 2 * kk + b

                @pl.when(k >= 2)
                def _():
                    wait_out(k - 2, b)

                wait_in(k, b)

                @pl.when(k + 2 < _UPW)
                def _():
                    issue_in(k + 2, b)

                issue_out(k, b)
            return carry

        lax.fori_loop(0, _UPW // 2, outer, 0)
        for b in range(2):
            wait_out(_UPW - 2 + b, b)

    @pl.when(tb != _NBLK - 1)
    def _():
        run()

    @pl.when(tb == _NBLK - 1)
    def _():
        run_tail()


def _lookup_body(table_hbm, idx_hbm, w_hbm, out_hbm,
                 idx_v, ihi2, poff_v, w_v,
                 rows0, rows1, rows2, rows3,
                 outb0, outb1, outb2, outb3,
                 gsem0, gsem1, gsem2, gsem3,
                 osem0, osem1, osem2, osem3):
    wid = lax.axis_index("s") * _NC + lax.axis_index("c")
    row_base = wid * _RPW
    b_base = wid * _BPW

    rows = (rows0, rows1, rows2, rows3)
    outs = (outb0, outb1, outb2, outb3)
    gsems = (gsem0, gsem1, gsem2, gsem3)
    osems = (osem0, osem1, osem2, osem3)

    pltpu.sync_copy(idx_hbm.at[pl.ds(row_base, _RPW)], idx_v)

    def split(g, carry):
        sl = pl.ds(g * _L, _L)
        iv = idx_v[sl]
        poff_v[sl] = lax.shift_left(jnp.bitwise_and(iv, 1), 6)
        return carry

    lax.fori_loop(0, _RPW // _L, split, 0)

    def split2(c2, carry):
        def inner(l, carry2):
            ihi2[c2, pl.ds(l * _L, _L)] = lax.shift_right_logical(
                idx_v[pl.ds(c2 * _CHUNK_R + l * _L, _L)], 1)
            return carry2
        return lax.fori_loop(0, _CHUNK_R // _L, inner, carry)

    lax.fori_loop(0, _NCH, split2, 0)

    gdesc = [None] * _NCH
    for p in range(_RING - 1):
        gdesc[p] = pltpu.async_copy(
            table_hbm.at[ihi2.at[p]], rows[p], gsems[p])

    pltpu.sync_copy(w_hbm.at[pl.ds(row_base, _RPW)], w_v)

    odesc = [None] * _NCH
    for c in range(_NCH):
        nxt = c + _RING - 1
        if nxt < _NCH:
            gdesc[nxt] = pltpu.async_copy(
                table_hbm.at[ihi2.at[nxt]], rows[nxt % _RING],
                gsems[nxt % _RING])
        gdesc[c].wait()
        if c >= _RING:
            odesc[c - _RING].wait()

        rbuf = rows[c % _RING]
        obuf = outs[c % _RING]
        woff = c * _CHUNK_R

        def body(g, carry, rbuf=rbuf, obuf=obuf, woff=woff):
            wsl = pl.ds(woff + g * _L, _L)
            wv = w_v[wsl]
            pv = poff_v[wsl]
            for j in range(_L // _COMP):
                e = g * (_L // _COMP) + j
                rb = e * _COMP
                o0 = pv[_COMP * j]
                o1 = pv[_COMP * j + 1]
                o2 = pv[_COMP * j + 2]
                o3 = pv[_COMP * j + 3]
                w0 = wv[_COMP * j]
                w1 = wv[_COMP * j + 1]
                w2 = wv[_COMP * j + 2]
                w3 = wv[_COMP * j + 3]
                for s in range(_DIMS // _L):
                    acc = (rbuf[rb, pl.ds(o0 + s * _L, _L)] * w0
                           + rbuf[rb + 1, pl.ds(o1 + s * _L, _L)] * w1
                           + rbuf[rb + 2, pl.ds(o2 + s * _L, _L)] * w2
                           + rbuf[rb + 3, pl.ds(o3 + s * _L, _L)] * w3)
                    obuf[e, pl.ds(s * _L, _L)] = acc
            return carry

        lax.fori_loop(0, _CHUNK_R // _L, body, 0)

        odesc[c] = pltpu.async_copy(
            obuf, out_hbm.at[pl.ds(b_base + c * _CHUNK_B, _CHUNK_B)],
            osems[c % _RING])

    for c in range(_NCH - _RING, _NCH):
        odesc[c].wait()


@jax.jit
def _sc_lookup(nat, tail, idx, w):
    mesh = plsc.VectorSubcoreMesh(core_axis_name="c", subcore_axis_name="s")
    relayout = pl.kernel(
        _relayout_body,
        out_type=jax.ShapeDtypeStruct((_NPAIR, _PAIR), jnp.float32),
        mesh=mesh,
        scratch_types=[
            pltpu.VMEM((_DIMS + 8, _PAIR), jnp.float32),
            pltpu.VMEM((_DIMS + 8, _PAIR), jnp.float32),
            pltpu.VMEM((_PAIR // 2, _PAIR), jnp.float32),
            pltpu.VMEM((_PAIR // 2, _PAIR), jnp.float32),
            pltpu.SemaphoreType.DMA,
            pltpu.SemaphoreType.DMA,
            pltpu.SemaphoreType.DMA,
            pltpu.SemaphoreType.DMA,
        ],
        compiler_params=pltpu.CompilerParams(needs_layout_passes=False),
    )
    table = relayout(nat, tail)

    lookup = pl.kernel(
        _lookup_body,
        out_type=jax.ShapeDtypeStruct((_BATCH, _DIMS), jnp.float32),
        mesh=mesh,
        scratch_types=[
            pltpu.VMEM((_RPW,), jnp.int32),
            pltpu.VMEM((_NCH, _CHUNK_R), jnp.int32),
            pltpu.VMEM((_RPW,), jnp.int32),
            pltpu.VMEM((_RPW,), jnp.float32),
            pltpu.VMEM((_CHUNK_R, _PAIR), jnp.float32),
            pltpu.VMEM((_CHUNK_R, _PAIR), jnp.float32),
            pltpu.VMEM((_CHUNK_R, _PAIR), jnp.float32),
            pltpu.VMEM((_CHUNK_R, _PAIR), jnp.float32),
            pltpu.VMEM((_CHUNK_B, _DIMS), jnp.float32),
            pltpu.VMEM((_CHUNK_B, _DIMS), jnp.float32),
            pltpu.VMEM((_CHUNK_B, _DIMS), jnp.float32),
            pltpu.VMEM((_CHUNK_B, _DIMS), jnp.float32),
            pltpu.SemaphoreType.DMA,
            pltpu.SemaphoreType.DMA,
            pltpu.SemaphoreType.DMA,
            pltpu.SemaphoreType.DMA,
            pltpu.SemaphoreType.DMA,
            pltpu.SemaphoreType.DMA,
            pltpu.SemaphoreType.DMA,
            pltpu.SemaphoreType.DMA,
        ],
    )
    return lookup(table, idx, w)


def kernel(coords, coord_weights, embeddings):
    nat = embeddings.transpose(0, 2, 1)
    tail = embeddings[:, _T1 - _PAIR:, :].reshape(_T0, _PAIR // 2, 2 * _DIMS)
    idx = (coords[..., 0].astype(jnp.int32) * _T1P
           + coords[..., 1].astype(jnp.int32)).reshape(-1)
    w = coord_weights.reshape(-1)
    return _sc_lookup(nat, tail, idx, w)


# speedup vs baseline: 1.6358x; 1.3501x over previous
import jax
import jax.numpy as jnp
from jax import lax
from jax.experimental import pallas as pl
from jax.experimental.pallas import tpu as pltpu
from jax.experimental.pallas import tpu_sc as plsc

_NC = 2
_NS = 16
_NW = _NC * _NS
_L = 16

_BATCH = 16384
_COMP = 4
_DIMS = 64
_PAIR = 128
_T0 = 1000
_T1 = 1000
_T1P = 1024
_NBLK = _T1P // _PAIR
_NPAIR = _T0 * _T1P // 2
_UPW = _T0 * _NBLK // _NW

_BPW = _BATCH // _NW
_RPW = _BPW * _COMP
_CHUNK_R = 128
_CHUNK_B = _CHUNK_R // _COMP
_NCH = _RPW // _CHUNK_R
_RING = 4


def _relayout_body(nat_hbm, tail_hbm, out_hbm, src0, src1, dst0, dst1,
                   ssem0, ssem1, dsem0, dsem1):
    wid = lax.axis_index("s") * _NC + lax.axis_index("c")
    tb = lax.shift_right_logical(wid, 2)
    t0w = jnp.bitwise_and(wid, 3)
    srcs = (src0, src1)
    dsts = (dst0, dst1)
    ssems = (ssem0, ssem1)
    dsems = (dsem0, dsem1)

    iotav = lax.iota(jnp.int32, _L)
    rowvecs = [lax.shift_right_logical(iotav + _L * t, 1)
               for t in range(_PAIR // _L)]
    cvec0 = lax.shift_left(jnp.bitwise_and(iotav, 1), 6)

    def unit_t0(k):
        return t0w + 4 * k

    def unit_out(k):
        return (unit_t0(k) * _NBLK + tb) * (_PAIR // 2)

    def run():
        npair = _PAIR // 2

        def src_slice(k):
            return nat_hbm.at[unit_t0(k), :, pl.ds(tb * _PAIR, _PAIR)]

        def issue_src(k, b):
            return pltpu.async_copy(src_slice(k), srcs[b].at[pl.ds(0, _DIMS)],
                                    ssems[b])

        def wait_src(k, b):
            pltpu.make_async_copy(src_slice(k), srcs[b].at[pl.ds(0, _DIMS)],
                                  ssems[b]).wait()

        def issue_dst(k, b):
            return pltpu.async_copy(
                dsts[b], out_hbm.at[pl.ds(unit_out(k), _PAIR // 2)],
                dsems[b])

        def wait_dst(k, b):
            pltpu.make_async_copy(
                dsts[b], out_hbm.at[pl.ds(unit_out(k), _PAIR // 2)],
                dsems[b]).wait()

        def transpose(b):
            sbuf = srcs[b]
            dbuf = dsts[b]

            @plsc.parallel_loop(0, _DIMS, 1, unroll=4)
            def _(d):
                colv = cvec0 + d
                for t in range(_PAIR // _L):
                    v = sbuf[d, pl.ds(t * _L, _L)]
                    plsc.store_scatter(dbuf, [rowvecs[t], colv], v)

        issue_src(0, 0)
        issue_src(1, 1)
        wait_src(0, 0)
        transpose(0)
        issue_src(2, 0)
        issue_dst(0, 0)
        wait_src(1, 1)
        transpose(1)
        issue_src(3, 1)
        issue_dst(1, 1)

        def outer(kk, carry):
            for b in range(2):
                k = 2 * kk + b
                wait_dst(k - 2, b)
                wait_src(k, b)
                transpose(b)
                issue_src(k + 2, b)
                issue_dst(k, b)
            return carry

        lax.fori_loop(1, _UPW // 2 - 1, outer, 0)

        for b in range(2):
            k = _UPW - 2 + b
            wait_dst(k - 2, b)
            wait_src(k, b)
            transpose(b)
            issue_dst(k, b)
        for b in range(2):
            wait_dst(_UPW - 2 + b, b)

    def run_tail():
        def issue_in(k, b):
            return pltpu.async_copy(tail_hbm.at[unit_t0(k)],
                                    srcs[b].at[pl.ds(0, _PAIR // 2)],
                                    ssems[b])

        def wait_in(k, b):
            pltpu.make_async_copy(tail_hbm.at[unit_t0(k)],
                                  srcs[b].at[pl.ds(0, _PAIR // 2)],
                                  ssems[b]).wait()

        def issue_out(k, b):
            return pltpu.async_copy(
                srcs[b].at[pl.ds(12, 56)],
                out_hbm.at[pl.ds(unit_out(k), 56)], dsems[b])

        def wait_out(k, b):
            pltpu.make_async_copy(
                srcs[b].at[pl.ds(12, 56)],
                out_hbm.at[pl.ds(unit_out(k), 56)], dsems[b]).wait()

        issue_in(0, 0)
        issue_in(1, 1)

        def outer(kk, carry):
            for b in range(2):
                k = 2 * kk + b

                @pl.when(k >= 2)
                def _():
                    wait_out(k - 2, b)

                wait_in(k, b)

                @pl.when(k + 2 < _UPW)
                def _():
                    issue_in(k + 2, b)

                issue_out(k, b)
            return carry

        lax.fori_loop(0, _UPW // 2, outer, 0)
        for b in range(2):
            wait_out(_UPW - 2 + b, b)

    @pl.when(tb != _NBLK - 1)
    def _():
        run()

    @pl.when(tb == _NBLK - 1)
    def _():
        run_tail()


def _lookup_body(table_hbm, idx_hbm, w_hbm, out_hbm,
                 idx_v, ihi2, poff_v, w_v,
                 rows0, rows1, rows2, rows3,
                 outb0, outb1, outb2, outb3,
                 gsem0, gsem1, gsem2, gsem3,
                 osem0, osem1, osem2, osem3):
    wid = lax.axis_index("s") * _NC + lax.axis_index("c")
    row_base = wid * _RPW
    b_base = wid * _BPW

    rows = (rows0, rows1, rows2, rows3)
    outs = (outb0, outb1, outb2, outb3)
    gsems = (gsem0, gsem1, gsem2, gsem3)
    osems = (osem0, osem1, osem2, osem3)

    pltpu.sync_copy(idx_hbm.at[pl.ds(row_base, _RPW)], idx_v)

    def split(g, carry):
        sl = pl.ds(g * _L, _L)
        iv = idx_v[sl]
        poff_v[sl] = lax.shift_left(jnp.bitwise_and(iv, 1), 6)
        return carry

    lax.fori_loop(0, _RPW // _L, split, 0)

    def split2(c2, carry):
        def inner(l, carry2):
            ihi2[c2, pl.ds(l * _L, _L)] = lax.shift_right_logical(
                idx_v[pl.ds(c2 * _CHUNK_R + l * _L, _L)], 1)
            return carry2
        return lax.fori_loop(0, _CHUNK_R // _L, inner, carry)

    lax.fori_loop(0, _NCH, split2, 0)

    gdesc = [None] * _NCH
    for p in range(_RING - 1):
        gdesc[p] = pltpu.async_copy(
            table_hbm.at[ihi2.at[p]], rows[p], gsems[p])

    pltpu.sync_copy(w_hbm.at[pl.ds(row_base, _RPW)], w_v)

    odesc = [None] * _NCH
    for c in range(_NCH):
        nxt = c + _RING - 1
        if nxt < _NCH:
            gdesc[nxt] = pltpu.async_copy(
                table_hbm.at[ihi2.at[nxt]], rows[nxt % _RING],
                gsems[nxt % _RING])
        gdesc[c].wait()
        if c >= _RING:
            odesc[c - _RING].wait()

        rbuf = rows[c % _RING]
        obuf = outs[c % _RING]
        woff = c * _CHUNK_R

        def body(g, carry, rbuf=rbuf, obuf=obuf, woff=woff):
            wsl = pl.ds(woff + g * _L, _L)
            wv = w_v[wsl]
            pv = poff_v[wsl]
            for j in range(_L // _COMP):
                e = g * (_L // _COMP) + j
                rb = e * _COMP
                o0 = pv[_COMP * j]
                o1 = pv[_COMP * j + 1]
                o2 = pv[_COMP * j + 2]
                o3 = pv[_COMP * j + 3]
                w0 = wv[_COMP * j]
                w1 = wv[_COMP * j + 1]
                w2 = wv[_COMP * j + 2]
                w3 = wv[_COMP * j + 3]
                for s in range(_DIMS // _L):
                    acc = (rbuf[rb, pl.ds(o0 + s * _L, _L)] * w0
                           + rbuf[rb + 1, pl.ds(o1 + s * _L, _L)] * w1
                           + rbuf[rb + 2, pl.ds(o2 + s * _L, _L)] * w2
                           + rbuf[rb + 3, pl.ds(o3 + s * _L, _L)] * w3)
                    obuf[e, pl.ds(s * _L, _L)] = acc
            return carry

        lax.fori_loop(0, _CHUNK_R // _L, body, 0)

        odesc[c] = pltpu.async_copy(
            obuf, out_hbm.at[pl.ds(b_base + c * _CHUNK_B, _CHUNK_B)],
            osems[c % _RING])

    for c in range(_NCH - _RING, _NCH):
        odesc[c].wait()


@jax.jit
def _sc_lookup(nat, tail, idx, w):
    mesh = plsc.VectorSubcoreMesh(core_axis_name="c", subcore_axis_name="s")
    relayout = pl.kernel(
        _relayout_body,
        out_type=jax.ShapeDtypeStruct((_NPAIR, _PAIR), jnp.float32),
        mesh=mesh,
        scratch_types=[
            pltpu.VMEM((_DIMS + 8, _PAIR), jnp.float32),
            pltpu.VMEM((_DIMS + 8, _PAIR), jnp.float32),
            pltpu.VMEM((_PAIR // 2, _PAIR), jnp.float32),
            pltpu.VMEM((_PAIR // 2, _PAIR), jnp.float32),
            pltpu.SemaphoreType.DMA,
            pltpu.SemaphoreType.DMA,
            pltpu.SemaphoreType.DMA,
            pltpu.SemaphoreType.DMA,
        ],
        compiler_params=pltpu.CompilerParams(needs_layout_passes=False),
    )
    table = relayout(nat, tail)

    lookup = pl.kernel(
        _lookup_body,
        out_type=jax.ShapeDtypeStruct((_BATCH, _DIMS), jnp.float32),
        mesh=mesh,
        scratch_types=[
            pltpu.VMEM((_RPW,), jnp.int32),
            pltpu.VMEM((_NCH, _CHUNK_R), jnp.int32),
            pltpu.VMEM((_RPW,), jnp.int32),
            pltpu.VMEM((_RPW,), jnp.float32),
            pltpu.VMEM((_CHUNK_R, _PAIR), jnp.float32),
            pltpu.VMEM((_CHUNK_R, _PAIR), jnp.float32),
            pltpu.VMEM((_CHUNK_R, _PAIR), jnp.float32),
            pltpu.VMEM((_CHUNK_R, _PAIR), jnp.float32),
            pltpu.VMEM((_CHUNK_B, _DIMS), jnp.float32),
            pltpu.VMEM((_CHUNK_B, _DIMS), jnp.float32),
            pltpu.VMEM((_CHUNK_B, _DIMS), jnp.float32),
            pltpu.VMEM((_CHUNK_B, _DIMS), jnp.float32),
            pltpu.SemaphoreType.DMA,
            pltpu.SemaphoreType.DMA,
            pltpu.SemaphoreType.DMA,
            pltpu.SemaphoreType.DMA,
            pltpu.SemaphoreType.DMA,
            pltpu.SemaphoreType.DMA,
            pltpu.SemaphoreType.DMA,
            pltpu.SemaphoreType.DMA,
        ],
    )
    return lookup(table, idx, w)


def kernel(coords, coord_weights, embeddings):
    nat = embeddings.transpose(0, 2, 1)
    tail = embeddings[:, _T1 - _PAIR:, :].reshape(_T0, _PAIR // 2, 2 * _DIMS)
    idx = (coords[..., 0].astype(jnp.int32) * _T1P
           + coords[..., 1].astype(jnp.int32)).reshape(-1)
    w = coord_weights.reshape(-1)
    return _sc_lookup(nat, tail, idx, w)


# speedup vs baseline: 1.7487x; 1.0690x over previous
import jax
import jax.numpy as jnp
from jax import lax
from jax.experimental import pallas as pl
from jax.experimental.pallas import tpu as pltpu
from jax.experimental.pallas import tpu_sc as plsc

_NC = 2
_NS = 16
_NW = _NC * _NS
_L = 16

_BATCH = 16384
_COMP = 4
_DIMS = 64
_PAIR = 128
_T0 = 1000
_T1 = 1000
_T1P = 1024
_NBLK = _T1P // _PAIR
_NPAIR = _T0 * _T1P // 2
_UPW = _T0 * _NBLK // _NW

_BPW = _BATCH // _NW
_RPW = _BPW * _COMP
_CHUNK_R = 128
_CHUNK_B = _CHUNK_R // _COMP
_NCH = _RPW // _CHUNK_R
_RING = 4


def _relayout_body(nat_hbm, tail_hbm, out_hbm, src0, src1, dst0, dst1,
                   tsrc0, tsrc1, ssem0, ssem1, dsem0, dsem1):
    wid = lax.axis_index("s") * _NC + lax.axis_index("c")
    tb = lax.shift_right_logical(wid, 2)
    t0w = jnp.bitwise_and(wid, 3)
    srcs = (src0, src1)
    dsts = (dst0, dst1)
    tsrcs = (tsrc0, tsrc1)
    ssems = (ssem0, ssem1)
    dsems = (dsem0, dsem1)

    iotav = lax.iota(jnp.int32, _L)
    dvecs = [iotav + _L * k4 for k4 in range(_DIMS // _L)]

    def unit_t0(k):
        return t0w + 4 * k

    def unit_out(k):
        return (unit_t0(k) * _NBLK + tb) * (_PAIR // 2)

    def run():
        npair = _PAIR // 2

        def src_slice(k):
            return nat_hbm.at[unit_t0(k), :, pl.ds(tb * _PAIR, _PAIR)]

        def issue_src(k, b):
            return pltpu.async_copy(src_slice(k),
                                    srcs[b].at[:, pl.ds(0, _PAIR)], ssems[b])

        def wait_src(k, b):
            pltpu.make_async_copy(src_slice(k),
                                  srcs[b].at[:, pl.ds(0, _PAIR)],
                                  ssems[b]).wait()

        def issue_dst(k, b):
            return pltpu.async_copy(
                dsts[b], out_hbm.at[pl.ds(unit_out(k), _PAIR // 2)],
                dsems[b])

        def wait_dst(k, b):
            pltpu.make_async_copy(
                dsts[b], out_hbm.at[pl.ds(unit_out(k), _PAIR // 2)],
                dsems[b]).wait()

        def transpose(b):
            sbuf = srcs[b]
            dbuf = dsts[b]

            @plsc.parallel_loop(0, _PAIR, 1, unroll=4)
            def _(t1):
                p = lax.shift_right_logical(t1, 1)
                h64 = lax.shift_left(jnp.bitwise_and(t1, 1), 6)
                cvec = jnp.full((_L,), t1, jnp.int32)
                for k4 in range(_DIMS // _L):
                    v = plsc.load_gather(sbuf, [dvecs[k4], cvec])
                    dbuf[p, pl.ds(h64 + k4 * _L, _L)] = v

        issue_src(0, 0)
        issue_src(1, 1)
        wait_src(0, 0)
        transpose(0)
        issue_src(2, 0)
        issue_dst(0, 0)
        wait_src(1, 1)
        transpose(1)
        issue_src(3, 1)
        issue_dst(1, 1)

        def outer(kk, carry):
            for b in range(2):
                k = 2 * kk + b
                wait_dst(k - 2, b)
                wait_src(k, b)
                transpose(b)
                issue_src(k + 2, b)
                issue_dst(k, b)
            return carry

        lax.fori_loop(1, _UPW // 2 - 1, outer, 0)

        for b in range(2):
            k = _UPW - 2 + b
            wait_dst(k - 2, b)
            wait_src(k, b)
            transpose(b)
            issue_dst(k, b)
        for b in range(2):
            wait_dst(_UPW - 2 + b, b)

    def run_tail():
        def issue_in(k, b):
            return pltpu.async_copy(tail_hbm.at[unit_t0(k)],
                                    tsrcs[b].at[pl.ds(0, _PAIR // 2)],
                                    ssems[b])

        def wait_in(k, b):
            pltpu.make_async_copy(tail_hbm.at[unit_t0(k)],
                                  tsrcs[b].at[pl.ds(0, _PAIR // 2)],
                                  ssems[b]).wait()

        def issue_out(k, b):
            return pltpu.async_copy(
                tsrcs[b].at[pl.ds(12, 56)],
                out_hbm.at[pl.ds(unit_out(k), 56)], dsems[b])

        def wait_out(k, b):
            pltpu.make_async_copy(
                tsrcs[b].at[pl.ds(12, 56)],
                out_hbm.at[pl.ds(unit_out(k), 56)], dsems[b]).wait()

        issue_in(0, 0)
        issue_in(1, 1)

        def outer(kk, carry):
            for b in range(2):
                k = 2 * kk + b

                @pl.when(k >= 2)
                def _():
                    wait_out(k - 2, b)

                wait_in(k, b)

                @pl.when(k + 2 < _UPW)
                def _():
                    issue_in(k + 2, b)

                issue_out(k, b)
            return carry

        lax.fori_loop(0, _UPW // 2, outer, 0)
        for b in range(2):
            wait_out(_UPW - 2 + b, b)

    @pl.when(tb != _NBLK - 1)
    def _():
        run()

    @pl.when(tb == _NBLK - 1)
    def _():
        run_tail()


def _lookup_body(table_hbm, idx_hbm, w_hbm, out_hbm,
                 idx_v, ihi2, poff_v, w_v,
                 rows0, rows1, rows2, rows3,
                 outb0, outb1, outb2, outb3,
                 gsem0, gsem1, gsem2, gsem3,
                 osem0, osem1, osem2, osem3):
    wid = lax.axis_index("s") * _NC + lax.axis_index("c")
    row_base = wid * _RPW
    b_base = wid * _BPW

    rows = (rows0, rows1, rows2, rows3)
    outs = (outb0, outb1, outb2, outb3)
    gsems = (gsem0, gsem1, gsem2, gsem3)
    osems = (osem0, osem1, osem2, osem3)

    pltpu.sync_copy(idx_hbm.at[pl.ds(row_base, _RPW)], idx_v)

    def split(g, carry):
        sl = pl.ds(g * _L, _L)
        iv = idx_v[sl]
        poff_v[sl] = lax.shift_left(jnp.bitwise_and(iv, 1), 6)
        return carry

    lax.fori_loop(0, _RPW // _L, split, 0)

    def split2(c2, carry):
        def inner(l, carry2):
            ihi2[c2, pl.ds(l * _L, _L)] = lax.shift_right_logical(
                idx_v[pl.ds(c2 * _CHUNK_R + l * _L, _L)], 1)
            return carry2
        return lax.fori_loop(0, _CHUNK_R // _L, inner, carry)

    lax.fori_loop(0, _NCH, split2, 0)

    gdesc = [None] * _NCH
    for p in range(_RING - 1):
        gdesc[p] = pltpu.async_copy(
            table_hbm.at[ihi2.at[p]], rows[p], gsems[p])

    pltpu.sync_copy(w_hbm.at[pl.ds(row_base, _RPW)], w_v)

    odesc = [None] * _NCH
    for c in range(_NCH):
        nxt = c + _RING - 1
        if nxt < _NCH:
            gdesc[nxt] = pltpu.async_copy(
                table_hbm.at[ihi2.at[nxt]], rows[nxt % _RING],
                gsems[nxt % _RING])
        gdesc[c].wait()
        if c >= _RING:
            odesc[c - _RING].wait()

        rbuf = rows[c % _RING]
        obuf = outs[c % _RING]
        woff = c * _CHUNK_R

        def body(g, carry, rbuf=rbuf, obuf=obuf, woff=woff):
            wsl = pl.ds(woff + g * _L, _L)
            wv = w_v[wsl]
            pv = poff_v[wsl]
            for j in range(_L // _COMP):
                e = g * (_L // _COMP) + j
                rb = e * _COMP
                o0 = pv[_COMP * j]
                o1 = pv[_COMP * j + 1]
                o2 = pv[_COMP * j + 2]
                o3 = pv[_COMP * j + 3]
                w0 = wv[_COMP * j]
                w1 = wv[_COMP * j + 1]
                w2 = wv[_COMP * j + 2]
                w3 = wv[_COMP * j + 3]
                for s in range(_DIMS // _L):
                    acc = (rbuf[rb, pl.ds(o0 + s * _L, _L)] * w0
                           + rbuf[rb + 1, pl.ds(o1 + s * _L, _L)] * w1
                           + rbuf[rb + 2, pl.ds(o2 + s * _L, _L)] * w2
                           + rbuf[rb + 3, pl.ds(o3 + s * _L, _L)] * w3)
                    obuf[e, pl.ds(s * _L, _L)] = acc
            return carry

        lax.fori_loop(0, _CHUNK_R // _L, body, 0)

        odesc[c] = pltpu.async_copy(
            obuf, out_hbm.at[pl.ds(b_base + c * _CHUNK_B, _CHUNK_B)],
            osems[c % _RING])

    for c in range(_NCH - _RING, _NCH):
        odesc[c].wait()


@jax.jit
def _sc_lookup(nat, tail, idx, w):
    mesh = plsc.VectorSubcoreMesh(core_axis_name="c", subcore_axis_name="s")
    relayout = pl.kernel(
        _relayout_body,
        out_type=jax.ShapeDtypeStruct((_NPAIR, _PAIR), jnp.float32),
        mesh=mesh,
        scratch_types=[
            pltpu.VMEM((_DIMS, _PAIR + 1), jnp.float32),
            pltpu.VMEM((_DIMS, _PAIR + 1), jnp.float32),
            pltpu.VMEM((_PAIR // 2, _PAIR), jnp.float32),
            pltpu.VMEM((_PAIR // 2, _PAIR), jnp.float32),
            pltpu.VMEM((_DIMS + 8, _PAIR), jnp.float32),
            pltpu.VMEM((_DIMS + 8, _PAIR), jnp.float32),
            pltpu.SemaphoreType.DMA,
            pltpu.SemaphoreType.DMA,
            pltpu.SemaphoreType.DMA,
            pltpu.SemaphoreType.DMA,
        ],
        compiler_params=pltpu.CompilerParams(needs_layout_passes=False),
    )
    table = relayout(nat, tail)

    lookup = pl.kernel(
        _lookup_body,
        out_type=jax.ShapeDtypeStruct((_BATCH, _DIMS), jnp.float32),
        mesh=mesh,
        scratch_types=[
            pltpu.VMEM((_RPW,), jnp.int32),
            pltpu.VMEM((_NCH, _CHUNK_R), jnp.int32),
            pltpu.VMEM((_RPW,), jnp.int32),
            pltpu.VMEM((_RPW,), jnp.float32),
            pltpu.VMEM((_CHUNK_R, _PAIR), jnp.float32),
            pltpu.VMEM((_CHUNK_R, _PAIR), jnp.float32),
            pltpu.VMEM((_CHUNK_R, _PAIR), jnp.float32),
            pltpu.VMEM((_CHUNK_R, _PAIR), jnp.float32),
            pltpu.VMEM((_CHUNK_B, _DIMS), jnp.float32),
            pltpu.VMEM((_CHUNK_B, _DIMS), jnp.float32),
            pltpu.VMEM((_CHUNK_B, _DIMS), jnp.float32),
            pltpu.VMEM((_CHUNK_B, _DIMS), jnp.float32),
            pltpu.SemaphoreType.DMA,
            pltpu.SemaphoreType.DMA,
            pltpu.SemaphoreType.DMA,
            pltpu.SemaphoreType.DMA,
            pltpu.SemaphoreType.DMA,
            pltpu.SemaphoreType.DMA,
            pltpu.SemaphoreType.DMA,
            pltpu.SemaphoreType.DMA,
        ],
    )
    return lookup(table, idx, w)


def kernel(coords, coord_weights, embeddings):
    nat = embeddings.transpose(0, 2, 1)
    tail = embeddings[:, _T1 - _PAIR:, :].reshape(_T0, _PAIR // 2, 2 * _DIMS)
    idx = (coords[..., 0].astype(jnp.int32) * _T1P
           + coords[..., 1].astype(jnp.int32)).reshape(-1)
    w = coord_weights.reshape(-1)
    return _sc_lookup(nat, tail, idx, w)


# speedup vs baseline: 2.6535x; 1.5174x over previous
import jax
import jax.numpy as jnp
from jax import lax
from jax.experimental import pallas as pl
from jax.experimental.pallas import tpu as pltpu
from jax.experimental.pallas import tpu_sc as plsc

_NC = 2
_NS = 16
_NW = _NC * _NS
_L = 16

_BATCH = 16384
_COMP = 4
_DIMS = 64
_PAIR = 128

_BPW = _BATCH // _NW
_RPW = _BPW * _COMP
_CHUNK_R = 128
_CHUNK_B = _CHUNK_R // _COMP
_NCH = _RPW // _CHUNK_R
_RING = 4


def _sc_body(table_hbm, idx_hbm, w_hbm, out_hbm,
             idx_v, ihi2, poff_v, w_v,
             rows0, rows1, rows2, rows3,
             outb0, outb1, outb2, outb3,
             gsem0, gsem1, gsem2, gsem3,
             osem0, osem1, osem2, osem3):
    wid = lax.axis_index("s") * _NC + lax.axis_index("c")
    row_base = wid * _RPW
    b_base = wid * _BPW

    rows = (rows0, rows1, rows2, rows3)
    outs = (outb0, outb1, outb2, outb3)
    gsems = (gsem0, gsem1, gsem2, gsem3)
    osems = (osem0, osem1, osem2, osem3)

    pltpu.sync_copy(idx_hbm.at[pl.ds(row_base, _RPW)], idx_v)

    def split(g, carry):
        sl = pl.ds(g * _L, _L)
        iv = idx_v[sl]
        poff_v[sl] = lax.shift_left(jnp.bitwise_and(iv, 1), 6)
        return carry

    lax.fori_loop(0, _RPW // _L, split, 0)

    def split2(c2, carry):
        def inner(l, carry2):
            ihi2[c2, pl.ds(l * _L, _L)] = lax.shift_right_logical(
                idx_v[pl.ds(c2 * _CHUNK_R + l * _L, _L)], 1)
            return carry2
        return lax.fori_loop(0, _CHUNK_R // _L, inner, carry)

    lax.fori_loop(0, _NCH, split2, 0)

    gdesc = [None] * _NCH
    for p in range(_RING - 1):
        gdesc[p] = pltpu.async_copy(
            table_hbm.at[ihi2.at[p]], rows[p], gsems[p])

    pltpu.sync_copy(w_hbm.at[pl.ds(row_base, _RPW)], w_v)

    odesc = [None] * _NCH
    for c in range(_NCH):
        nxt = c + _RING - 1
        if nxt < _NCH:
            gdesc[nxt] = pltpu.async_copy(
                table_hbm.at[ihi2.at[nxt]], rows[nxt % _RING],
                gsems[nxt % _RING])
        gdesc[c].wait()
        if c >= _RING:
            odesc[c - _RING].wait()

        rbuf = rows[c % _RING]
        obuf = outs[c % _RING]
        woff = c * _CHUNK_R

        def body(g, carry, rbuf=rbuf, obuf=obuf, woff=woff):
            wsl = pl.ds(woff + g * _L, _L)
            wv = w_v[wsl]
            pv = poff_v[wsl]
            for j in range(_L // _COMP):
                e = g * (_L // _COMP) + j
                rb = e * _COMP
                o0 = pv[_COMP * j]
                o1 = pv[_COMP * j + 1]
                o2 = pv[_COMP * j + 2]
                o3 = pv[_COMP * j + 3]
                w0 = wv[_COMP * j]
                w1 = wv[_COMP * j + 1]
                w2 = wv[_COMP * j + 2]
                w3 = wv[_COMP * j + 3]
                for s in range(_DIMS // _L):
                    acc = (rbuf[rb, pl.ds(o0 + s * _L, _L)] * w0
                           + rbuf[rb + 1, pl.ds(o1 + s * _L, _L)] * w1
                           + rbuf[rb + 2, pl.ds(o2 + s * _L, _L)] * w2
                           + rbuf[rb + 3, pl.ds(o3 + s * _L, _L)] * w3)
                    obuf[e, pl.ds(s * _L, _L)] = acc
            return carry

        lax.fori_loop(0, _CHUNK_R // _L, body, 0)

        odesc[c] = pltpu.async_copy(
            obuf, out_hbm.at[pl.ds(b_base + c * _CHUNK_B, _CHUNK_B)],
            osems[c % _RING])

    for c in range(_NCH - _RING, _NCH):
        odesc[c].wait()


@jax.jit
def _sc_lookup(table, idx, w):
    mesh = plsc.VectorSubcoreMesh(core_axis_name="c", subcore_axis_name="s")
    k = pl.kernel(
        _sc_body,
        out_type=jax.ShapeDtypeStruct((_BATCH, _DIMS), jnp.float32),
        mesh=mesh,
        scratch_types=[
            pltpu.VMEM((_RPW,), jnp.int32),
            pltpu.VMEM((_NCH, _CHUNK_R), jnp.int32),
            pltpu.VMEM((_RPW,), jnp.int32),
            pltpu.VMEM((_RPW,), jnp.float32),
            pltpu.VMEM((_CHUNK_R, _PAIR), jnp.float32),
            pltpu.VMEM((_CHUNK_R, _PAIR), jnp.float32),
            pltpu.VMEM((_CHUNK_R, _PAIR), jnp.float32),
            pltpu.VMEM((_CHUNK_R, _PAIR), jnp.float32),
            pltpu.VMEM((_CHUNK_B, _DIMS), jnp.float32),
            pltpu.VMEM((_CHUNK_B, _DIMS), jnp.float32),
            pltpu.VMEM((_CHUNK_B, _DIMS), jnp.float32),
            pltpu.VMEM((_CHUNK_B, _DIMS), jnp.float32),
            pltpu.SemaphoreType.DMA,
            pltpu.SemaphoreType.DMA,
            pltpu.SemaphoreType.DMA,
            pltpu.SemaphoreType.DMA,
            pltpu.SemaphoreType.DMA,
            pltpu.SemaphoreType.DMA,
            pltpu.SemaphoreType.DMA,
            pltpu.SemaphoreType.DMA,
        ],
    )
    return k(table, idx, w)


def kernel(coords, coord_weights, embeddings):
    t1 = embeddings.shape[1]
    dims = embeddings.shape[-1]
    idx = (coords[..., 0].astype(jnp.int32) * t1
           + coords[..., 1].astype(jnp.int32)).reshape(-1)
    w = coord_weights.reshape(-1)
    table = embeddings.reshape(-1, 2 * dims)
    return _sc_lookup(table, idx, w)
